# baseline (device time: 32070 ns/iter reference)
import jax
import jax.numpy as jnp
from jax import lax
from jax.experimental import pallas as pl
from jax.experimental.pallas import tpu as pltpu

N_DEV = 4
B, Sq, Skv, Hq, Dh = 2, 128, 128, 16, 64
H_LOC = Hq // N_DEV
D_LOC = H_LOC * Dh
D_MODEL = 512
NEG_INF = -1e9


def _body(x_ref, wq_ref, k_ref, v_ref, wo_ref, out_ref,
          comm_ref, send_sems, recv_sems):
    my_pos = lax.axis_index("i")
    left = lax.rem(my_pos + N_DEV - 1, N_DEV)
    right = lax.rem(my_pos + 1, N_DEV)

    barrier_sem = pltpu.get_barrier_semaphore()
    for nbr in (left, right):
        pl.semaphore_signal(
            barrier_sem, inc=1,
            device_id=(nbr,), device_id_type=pl.DeviceIdType.MESH,
        )
    pl.semaphore_wait(barrier_sem, 2)

    x = x_ref[...]
    q_all = jnp.dot(x, wq_ref[...],
                    preferred_element_type=jnp.float32)

    qb = lax.broadcasted_iota(jnp.int32, (Sq, Skv), 0) // 64
    kb = lax.broadcasted_iota(jnp.int32, (Sq, Skv), 1) // 64
    mask = (qb == kb) | (kb == 0) | (lax.rem(qb + kb, 3) == 0)

    rows = []
    for b in range(B):
        ctx_h = []
        for h in range(H_LOC):
            q = q_all[b * Sq:(b + 1) * Sq, h * Dh:(h + 1) * Dh]
            k = k_ref[b * Skv:(b + 1) * Skv, h * Dh:(h + 1) * Dh]
            s = lax.dot_general(
                q, k, (((1,), (1,)), ((), ())),
                preferred_element_type=jnp.float32) * 0.125
            s = jnp.where(mask, s, NEG_INF)
            s = s - jnp.max(s, axis=-1, keepdims=True)
            e = jnp.exp(s)
            w = e / jnp.sum(e, axis=-1, keepdims=True)
            v = v_ref[b * Skv:(b + 1) * Skv, h * Dh:(h + 1) * Dh]
            ctx_h.append(jnp.dot(w, v, preferred_element_type=jnp.float32))
        ctx_b = jnp.concatenate(ctx_h, axis=1)
        rows.append(jnp.dot(ctx_b, wo_ref[...],
                            preferred_element_type=jnp.float32))
    partial = jnp.concatenate(rows, axis=0)

    comm_ref[0] = partial
    acc = partial
    for hop in range(N_DEV - 1):
        send_slot = hop % 2
        recv_slot = (hop + 1) % 2
        rdma = pltpu.make_async_remote_copy(
            src_ref=comm_ref.at[send_slot],
            dst_ref=comm_ref.at[recv_slot],
            send_sem=send_sems.at[send_slot],
            recv_sem=recv_sems.at[recv_slot],
            device_id=(right,),
            device_id_type=pl.DeviceIdType.MESH,
        )
        rdma.start()
        rdma.wait()
        acc = acc + comm_ref[recv_slot]
    out_ref[...] = acc


def kernel(x, Wq, K_ext, V_ext, Wo):
    my = lax.axis_index("i")
    h0 = my * H_LOC
    K_loc = lax.dynamic_slice_in_dim(K_ext, h0, H_LOC, axis=2)
    V_loc = lax.dynamic_slice_in_dim(V_ext, h0, H_LOC, axis=2)
    x2 = x.reshape(B * Sq, D_MODEL)
    k2 = K_loc.reshape(B * Skv, D_LOC)
    v2 = V_loc.reshape(B * Skv, D_LOC)
    out = pl.pallas_call(
        _body,
        out_shape=jax.ShapeDtypeStruct((B * Sq, D_MODEL), jnp.float32),
        in_specs=[pl.BlockSpec(memory_space=pltpu.VMEM)] * 5,
        out_specs=pl.BlockSpec(memory_space=pltpu.VMEM),
        scratch_shapes=[
            pltpu.VMEM((2, B * Sq, D_MODEL), jnp.float32),
            pltpu.SemaphoreType.DMA((2,)),
            pltpu.SemaphoreType.DMA((2,)),
        ],
        compiler_params=pltpu.CompilerParams(collective_id=0),
    )(x2, Wq, k2, v2, Wo)
    return out.reshape(B, Sq, D_MODEL)


# device time: 18917 ns/iter; 1.6953x vs baseline; 1.6953x over previous
import jax
import jax.numpy as jnp
from jax import lax
from jax.experimental import pallas as pl
from jax.experimental.pallas import tpu as pltpu

N_DEV = 4
B, Sq, Skv, Hq, Dh = 2, 128, 128, 16, 64
H_LOC = Hq // N_DEV
D_LOC = H_LOC * Dh
D_MODEL = 512
NEG_INF = -1e9


def _body(x_ref, wq_ref, k_ref, v_ref, wo_ref, out_ref,
          comm_ref, send_sems, recv_sems):
    my_pos = lax.axis_index("i")
    p_a = my_pos ^ 1
    p_b = (N_DEV - 1) - my_pos

    barrier_sem = pltpu.get_barrier_semaphore()
    for nbr in (p_a, p_b):
        pl.semaphore_signal(
            barrier_sem, inc=1,
            device_id=(nbr,), device_id_type=pl.DeviceIdType.MESH,
        )
    pl.semaphore_wait(barrier_sem, 2)

    qb = lax.broadcasted_iota(jnp.int32, (Sq, Skv), 0) // 64
    kb = lax.broadcasted_iota(jnp.int32, (Sq, Skv), 1) // 64
    mask = (qb == kb) | (kb == 0) | (lax.rem(qb + kb, 3) == 0)

    def partial_for_batch(b):
        xb = x_ref[b * Sq:(b + 1) * Sq, :]
        q_b = jnp.dot(xb, wq_ref[...],
                      preferred_element_type=jnp.float32)
        ctx_h = []
        for h in range(H_LOC):
            q = q_b[:, h * Dh:(h + 1) * Dh]
            k = k_ref[b * Skv:(b + 1) * Skv, h * Dh:(h + 1) * Dh]
            s = lax.dot_general(
                q, k, (((1,), (1,)), ((), ())),
                preferred_element_type=jnp.float32) * 0.125
            s = jnp.where(mask, s, NEG_INF)
            s = s - jnp.max(s, axis=-1, keepdims=True)
            e = jnp.exp(s)
            w = e / jnp.sum(e, axis=-1, keepdims=True)
            v = v_ref[b * Skv:(b + 1) * Skv, h * Dh:(h + 1) * Dh]
            ctx_h.append(jnp.dot(w, v, preferred_element_type=jnp.float32))
        ctx_b = jnp.concatenate(ctx_h, axis=1)
        return jnp.dot(ctx_b, wo_ref[...],
                       preferred_element_type=jnp.float32)

    def exchange(half, partner, slot):
        return pltpu.make_async_remote_copy(
            src_ref=out_ref.at[pl.ds(half * Sq, Sq)],
            dst_ref=comm_ref.at[slot],
            send_sem=send_sems.at[slot],
            recv_sem=recv_sems.at[slot],
            device_id=(partner,),
            device_id_type=pl.DeviceIdType.MESH,
        )

    out_ref[0:Sq, :] = partial_for_batch(0)
    r1h0 = exchange(0, p_a, 0)
    r1h0.start()

    out_ref[Sq:2 * Sq, :] = partial_for_batch(1)
    r1h1 = exchange(1, p_b, 1)
    r1h1.start()

    r1h0.wait()
    out_ref[0:Sq, :] = out_ref[0:Sq, :] + comm_ref[0]
    r2h0 = exchange(0, p_b, 2)
    r2h0.start()

    r1h1.wait()
    out_ref[Sq:2 * Sq, :] = out_ref[Sq:2 * Sq, :] + comm_ref[1]
    r2h1 = exchange(1, p_a, 3)
    r2h1.start()

    r2h0.wait()
    out_ref[0:Sq, :] = out_ref[0:Sq, :] + comm_ref[2]
    r2h1.wait()
    out_ref[Sq:2 * Sq, :] = out_ref[Sq:2 * Sq, :] + comm_ref[3]


def kernel(x, Wq, K_ext, V_ext, Wo):
    my = lax.axis_index("i")
    h0 = my * H_LOC
    K_loc = lax.dynamic_slice_in_dim(K_ext, h0, H_LOC, axis=2)
    V_loc = lax.dynamic_slice_in_dim(V_ext, h0, H_LOC, axis=2)
    x2 = x.reshape(B * Sq, D_MODEL)
    k2 = K_loc.reshape(B * Skv, D_LOC)
    v2 = V_loc.reshape(B * Skv, D_LOC)
    out = pl.pallas_call(
        _body,
        out_shape=jax.ShapeDtypeStruct((B * Sq, D_MODEL), jnp.float32),
        in_specs=[pl.BlockSpec(memory_space=pltpu.VMEM)] * 5,
        out_specs=pl.BlockSpec(memory_space=pltpu.VMEM),
        scratch_shapes=[
            pltpu.VMEM((4, Sq, D_MODEL), jnp.float32),
            pltpu.SemaphoreType.DMA((4,)),
            pltpu.SemaphoreType.DMA((4,)),
        ],
        compiler_params=pltpu.CompilerParams(collective_id=0),
    )(x2, Wq, k2, v2, Wo)
    return out.reshape(B, Sq, D_MODEL)
